# baseline (speedup 1.0000x reference)
import jax
import jax.numpy as jnp
from jax import lax
from jax.experimental import pallas as pl
from jax.experimental.pallas import tpu as pltpu
from jax.experimental.pallas import tpu_sc as plsc

_NUM_LAYERS = 64
_OUT_W = 16
_OUT_H = 16
_BOX = _NUM_LAYERS * _OUT_W * _OUT_H
_T = 200
_G = (_T + 15) // 16
_NW = 32
_RC = 8

_MESH = dict(core_axis_name="c", subcore_axis_name="s", num_cores=2,
             num_subcores=16)


def _decode_body(tok_hbm, keys_hbm, vals_hbm,
                 in0, ok0, ov0, in1, ok1, ov1, sin0, sout0, sin1, sout1):
    wid = lax.axis_index("s") * 2 + lax.axis_index("c")
    rows = keys_hbm.shape[0] // (_NW * _T)
    base_row = wid * rows
    chunks = rows // _RC
    lanes = lax.iota(jnp.int32, 16)

    def _prefetch(c, in_v, sin):
        pltpu.async_copy(
            tok_hbm.at[pl.ds((base_row + c * _RC) * (_T * 3), _RC * _T * 3)],
            in_v, sin)

    def _chunk(c, in_v, ok_v, ov_v, sin, sout):
        crow = base_row + c * _RC
        pltpu.make_async_copy(
            tok_hbm.at[pl.ds(crow * (_T * 3), _RC * _T * 3)], in_v, sin
        ).wait()

        @pl.when(c >= 2)
        def _():
            pltpu.make_async_copy(
                ok_v.at[pl.ds(0, _RC * _T)],
                keys_hbm.at[pl.ds(crow * _T, _RC * _T)], sout).wait()
            pltpu.make_async_copy(
                ov_v.at[pl.ds(0, _RC * _T)],
                vals_hbm.at[pl.ds(crow * _T, _RC * _T)], sout).wait()

        for rloc in range(_RC):
            for g in [_G - 1] + list(range(_G - 1)):
                pos = lanes + (g * 16)
                pidx = jnp.minimum(pos, _T - 1) * 3 + rloc * (_T * 3)
                obs0 = plsc.load_gather(in_v, [pidx])
                atr = plsc.load_gather(in_v, [pidx + 1])
                val = plsc.load_gather(in_v, [pidx + 2])
                coords = obs0 & 255
                valid = (coords != 255) & (atr < _NUM_LAYERS)
                sidx = jnp.where(valid, atr * 256 + coords, 0)
                sval = jnp.where(valid, val.astype(jnp.float32), 0.0)
                ok_v[pl.ds(rloc * _T + g * 16, 16)] = (
                    sidx + (crow + rloc) * _BOX)
                ov_v[pl.ds(rloc * _T + g * 16, 16)] = sval

        pltpu.async_copy(ok_v.at[pl.ds(0, _RC * _T)],
                         keys_hbm.at[pl.ds(crow * _T, _RC * _T)], sout)
        pltpu.async_copy(ov_v.at[pl.ds(0, _RC * _T)],
                         vals_hbm.at[pl.ds(crow * _T, _RC * _T)], sout)

        @pl.when(c + 2 < chunks)
        def _():
            _prefetch(c + 2, in_v, sin)

    _prefetch(0, in0, sin0)
    _prefetch(1, in1, sin1)

    def _pair(cc, _):
        _chunk(2 * cc, in0, ok0, ov0, sin0, sout0)
        _chunk(2 * cc + 1, in1, ok1, ov1, sin1, sout1)
        return 0
    lax.fori_loop(0, chunks // 2, _pair, 0)

    for ok_v, ov_v, sout in ((ok0, ov0, sout0), (ok1, ov1, sout1)):
        pltpu.make_async_copy(
            ok_v.at[pl.ds(0, _RC * _T)],
            keys_hbm.at[pl.ds(base_row * _T, _RC * _T)], sout).wait()
        pltpu.make_async_copy(
            ov_v.at[pl.ds(0, _RC * _T)],
            vals_hbm.at[pl.ds(base_row * _T, _RC * _T)], sout).wait()


def _scatter_body(sk_hbm, sv_hbm, out_hbm,
                  k0v, v0v, t0v, box0, k1v, v1v, t1v, box1,
                  sin0, sout0, sin1, sout1):
    wid = lax.axis_index("s") * 2 + lax.axis_index("c")
    b_tt = out_hbm.shape[0]
    rows = b_tt // _NW
    base_row = wid * rows
    zero16 = jnp.zeros((16,), jnp.float32)
    zero16i = jnp.zeros((16,), jnp.int32)
    neg1 = jnp.full((16,), -1, jnp.int32)

    def _clear(j, _):
        box0[pl.ds(j * 16, 16)] = zero16
        box1[pl.ds(j * 16, 16)] = zero16
        return 0
    lax.fori_loop(0, _BOX // 16, _clear, 0)
    for g in range(_G):
        t0v[pl.ds(g * 16, 16)] = zero16i
        t1v[pl.ds(g * 16, 16)] = zero16i
    k0v[pl.ds(_T, 16)] = neg1
    k1v[pl.ds(_T, 16)] = neg1

    def _fetch(row, kv, vv, sin):
        pltpu.async_copy(sk_hbm.at[pl.ds(row * _T, _T)],
                         kv.at[pl.ds(0, _T)], sin)
        pltpu.async_copy(sv_hbm.at[pl.ds(row * _T, _T)],
                         vv.at[pl.ds(0, _T)], sin)

    def _row(r, kv, vv, tv, box, sin, sout):
        row = base_row + r
        pltpu.make_async_copy(sk_hbm.at[pl.ds(row * _T, _T)],
                              kv.at[pl.ds(0, _T)], sin).wait()
        pltpu.make_async_copy(sv_hbm.at[pl.ds(row * _T, _T)],
                              vv.at[pl.ds(0, _T)], sin).wait()

        @pl.when(r >= 2)
        def _():
            pltpu.make_async_copy(box, out_hbm.at[row], sout).wait()

        for g in range(_G):
            idx = tv[pl.ds(g * 16, 16)]
            plsc.store_scatter(box, [idx], zero16)
        for g in range(_G):
            k0 = kv[pl.ds(g * 16, 16)]
            k1 = kv[pl.ds(g * 16 + 1, 16)]
            v0 = vv[pl.ds(g * 16, 16)]
            idx = k0 & (_BOX - 1)
            m = (k0 != k1) & (k0 > -1)
            plsc.store_scatter(box, [idx], v0, mask=m)
            tv[pl.ds(g * 16, 16)] = idx

        pltpu.async_copy(box, out_hbm.at[row], sout)

        @pl.when(r + 2 < rows)
        def _():
            _fetch(row + 2, kv, vv, sin)

    _fetch(base_row, k0v, v0v, sin0)
    _fetch(base_row + 1, k1v, v1v, sin1)

    def _pair(rr, _):
        _row(2 * rr, k0v, v0v, t0v, box0, sin0, sout0)
        _row(2 * rr + 1, k1v, v1v, t1v, box1, sin1, sout1)
        return 0
    lax.fori_loop(0, rows // 2, _pair, 0)

    pltpu.make_async_copy(box0, out_hbm.at[base_row], sout0).wait()
    pltpu.make_async_copy(box1, out_hbm.at[base_row], sout1).wait()


def kernel(token_observations):
    b_tt = token_observations.shape[0]
    tok_flat = token_observations.reshape(b_tt * _T * 3)

    decode = pl.kernel(
        _decode_body,
        out_type=(
            jax.ShapeDtypeStruct((b_tt * _T,), jnp.int32),
            jax.ShapeDtypeStruct((b_tt * _T,), jnp.float32),
        ),
        mesh=plsc.VectorSubcoreMesh(**_MESH),
        scratch_types=[
            pltpu.VMEM((_RC * _T * 3,), jnp.int32),
            pltpu.VMEM((_RC * _T + 16,), jnp.int32),
            pltpu.VMEM((_RC * _T + 16,), jnp.float32),
            pltpu.VMEM((_RC * _T * 3,), jnp.int32),
            pltpu.VMEM((_RC * _T + 16,), jnp.int32),
            pltpu.VMEM((_RC * _T + 16,), jnp.float32),
            pltpu.SemaphoreType.DMA,
            pltpu.SemaphoreType.DMA,
            pltpu.SemaphoreType.DMA,
            pltpu.SemaphoreType.DMA,
        ],
        compiler_params=pltpu.CompilerParams(needs_layout_passes=False),
    )
    keys1d, vals1d = decode(tok_flat)

    sk, sv = lax.sort(
        (keys1d, vals1d), dimension=0, num_keys=1, is_stable=False,
    )

    scatter = pl.kernel(
        _scatter_body,
        out_type=jax.ShapeDtypeStruct((b_tt, _BOX), jnp.float32),
        mesh=plsc.VectorSubcoreMesh(**_MESH),
        scratch_types=[
            pltpu.VMEM((_T + 16,), jnp.int32),
            pltpu.VMEM((_G * 16,), jnp.float32),
            pltpu.VMEM((_G * 16,), jnp.int32),
            pltpu.VMEM((_BOX,), jnp.float32),
            pltpu.VMEM((_T + 16,), jnp.int32),
            pltpu.VMEM((_G * 16,), jnp.float32),
            pltpu.VMEM((_G * 16,), jnp.int32),
            pltpu.VMEM((_BOX,), jnp.float32),
            pltpu.SemaphoreType.DMA,
            pltpu.SemaphoreType.DMA,
            pltpu.SemaphoreType.DMA,
            pltpu.SemaphoreType.DMA,
        ],
        compiler_params=pltpu.CompilerParams(needs_layout_passes=False),
    )
    out = scatter(sk, sv)
    return out.reshape(b_tt, _NUM_LAYERS, _OUT_W, _OUT_H)
